# initial kernel scaffold (unmeasured)
import jax
import jax.numpy as jnp
from jax import lax
from jax.experimental import pallas as pl
from jax.experimental.pallas import tpu as pltpu

N_DEV = 4
B, SQ, D_MODEL = 2, 256, 512
SKV = 1024
HQ, DH = 16, 64
H_LOC = HQ // N_DEV
SKV_LOC = SKV // N_DEV


def kernel(x, Wq, K_ext, V_ext, Wo):
    def body(x_ref, wq_ref, k_ref, v_ref, wo_ref, out_ref,
             ksend, vsend, krecv, vrecv, precv,
             k_send_sems, k_recv_sems, v_send_sems, v_recv_sems,
             p_send_sems, p_recv_sems):
        my = lax.axis_index("i")

        barrier = pltpu.get_barrier_semaphore()
        for d in range(1, N_DEV):
            peer = lax.rem(my + d, N_DEV)
            pl.semaphore_signal(barrier, inc=1, device_id=(peer,),
                                device_id_type=pl.DeviceIdType.MESH)
        pl.semaphore_wait(barrier, N_DEV - 1)

        k_t = jnp.transpose(k_ref[...], (2, 0, 1, 3))
        v_t = jnp.transpose(v_ref[...], (2, 0, 1, 3))
        ksend[...] = k_t
        vsend[...] = v_t
        krecv[pl.ds(my, 1)] = lax.dynamic_slice_in_dim(k_t, my * H_LOC, H_LOC, 0)[None]
        vrecv[pl.ds(my, 1)] = lax.dynamic_slice_in_dim(v_t, my * H_LOC, H_LOC, 0)[None]

        kv_rdmas = []
        for d in range(1, N_DEV):
            j = lax.rem(my + d, N_DEV)
            for send_buf, recv_buf, ssems, rsems in (
                (ksend, krecv, k_send_sems, k_recv_sems),
                (vsend, vrecv, v_send_sems, v_recv_sems),
            ):
                r = pltpu.make_async_remote_copy(
                    src_ref=send_buf.at[pl.ds(j * H_LOC, H_LOC)],
                    dst_ref=recv_buf.at[my],
                    send_sem=ssems.at[j],
                    recv_sem=rsems.at[my],
                    device_id=(j,),
                    device_id_type=pl.DeviceIdType.MESH,
                )
                r.start()
                kv_rdmas.append(r)

        wq = wq_ref[...]
        qs = [lax.dot(x_ref[b], wq, preferred_element_type=jnp.float32)
              for b in range(B)]

        qmod = lax.broadcasted_iota(jnp.int32, (SQ, SKV), 0) // 64 % 4
        kmod = lax.broadcasted_iota(jnp.int32, (SQ, SKV), 1) // 64 % 4
        mask = qmod == kmod

        for d in range(1, N_DEV):
            j = lax.rem(my + d, N_DEV)
            for send_buf, recv_buf, ssems, rsems in (
                (ksend, krecv, k_send_sems, k_recv_sems),
                (vsend, vrecv, v_send_sems, v_recv_sems),
            ):
                pltpu.make_async_remote_copy(
                    src_ref=send_buf.at[pl.ds(0, H_LOC)],
                    dst_ref=recv_buf.at[j],
                    send_sem=ssems.at[j],
                    recv_sem=rsems.at[j],
                    device_id=(j,),
                    device_id_type=pl.DeviceIdType.MESH,
                ).wait_recv()

        wo = wo_ref[...]
        partial_bs = []
        for b in range(B):
            head_ctx = []
            for h in range(H_LOC):
                q_bh = qs[b][:, h * DH:(h + 1) * DH]
                k_bh = jnp.concatenate(
                    [krecv[s, h, b] for s in range(N_DEV)], axis=0)
                v_bh = jnp.concatenate(
                    [vrecv[s, h, b] for s in range(N_DEV)], axis=0)
                s_ = lax.dot_general(
                    q_bh, k_bh, (((1,), (1,)), ((), ())),
                    preferred_element_type=jnp.float32) * 0.125
                s_ = jnp.where(mask, s_, -1e9)
                m_ = jnp.max(s_, axis=1, keepdims=True)
                e_ = jnp.exp(s_ - m_)
                p_ = e_ / jnp.sum(e_, axis=1, keepdims=True)
                head_ctx.append(
                    lax.dot(p_, v_bh, preferred_element_type=jnp.float32))
            ctx_b = jnp.concatenate(head_ctx, axis=1)
            partial_bs.append(
                lax.dot(ctx_b, wo, preferred_element_type=jnp.float32))
        partial = jnp.stack(partial_bs, axis=0)

        precv[pl.ds(my, 1)] = partial[None]
        p_rdmas = []
        for d in range(1, N_DEV):
            j = lax.rem(my + d, N_DEV)
            r = pltpu.make_async_remote_copy(
                src_ref=precv.at[my],
                dst_ref=precv.at[my],
                send_sem=p_send_sems.at[j],
                recv_sem=p_recv_sems.at[my],
                device_id=(j,),
                device_id_type=pl.DeviceIdType.MESH,
            )
            r.start()
            p_rdmas.append(r)
        for d in range(1, N_DEV):
            j = lax.rem(my + d, N_DEV)
            pltpu.make_async_remote_copy(
                src_ref=precv.at[my],
                dst_ref=precv.at[j],
                send_sem=p_send_sems.at[j],
                recv_sem=p_recv_sems.at[j],
                device_id=(j,),
                device_id_type=pl.DeviceIdType.MESH,
            ).wait_recv()

        out_ref[...] = jnp.sum(precv[...], axis=0)

        for r in kv_rdmas + p_rdmas:
            r.wait_send()

    return pl.pallas_call(
        body,
        out_shape=jax.ShapeDtypeStruct((B, SQ, D_MODEL), jnp.float32),
        in_specs=[pl.BlockSpec(memory_space=pltpu.VMEM)] * 5,
        out_specs=pl.BlockSpec(memory_space=pltpu.VMEM),
        scratch_shapes=[
            pltpu.VMEM((HQ, B, SKV_LOC, DH), jnp.float32),
            pltpu.VMEM((HQ, B, SKV_LOC, DH), jnp.float32),
            pltpu.VMEM((N_DEV, H_LOC, B, SKV_LOC, DH), jnp.float32),
            pltpu.VMEM((N_DEV, H_LOC, B, SKV_LOC, DH), jnp.float32),
            pltpu.VMEM((N_DEV, B, SQ, D_MODEL), jnp.float32),
            pltpu.SemaphoreType.DMA((N_DEV,)),
            pltpu.SemaphoreType.DMA((N_DEV,)),
            pltpu.SemaphoreType.DMA((N_DEV,)),
            pltpu.SemaphoreType.DMA((N_DEV,)),
            pltpu.SemaphoreType.DMA((N_DEV,)),
            pltpu.SemaphoreType.DMA((N_DEV,)),
        ],
        compiler_params=pltpu.CompilerParams(collective_id=0),
    )(x, Wq, K_ext, V_ext, Wo)


# baseline (device time: 90142 ns/iter reference)
import jax
import jax.numpy as jnp
from jax import lax
from jax.experimental import pallas as pl
from jax.experimental.pallas import tpu as pltpu

N_DEV = 4
B, SQ, D_MODEL = 2, 256, 512
SKV = 1024
HQ, DH = 16, 64
H_LOC = HQ // N_DEV
SKV_LOC = SKV // N_DEV


def kernel(x, Wq, K_ext, V_ext, Wo):
    def body(x_ref, wq_ref, k_ref, v_ref, wo_ref, out_ref,
             ksend, vsend, krecv, vrecv, precv,
             k_send_sems, k_recv_sems, v_send_sems, v_recv_sems,
             p_send_sems, p_recv_sems):
        my = lax.axis_index("i")

        barrier = pltpu.get_barrier_semaphore()
        for d in range(1, N_DEV):
            peer = lax.rem(my + d, N_DEV)
            pl.semaphore_signal(barrier, inc=1, device_id=(peer,),
                                device_id_type=pl.DeviceIdType.MESH)
        pl.semaphore_wait(barrier, N_DEV - 1)

        k_t = jnp.transpose(k_ref[...], (2, 0, 1, 3))
        v_t = jnp.transpose(v_ref[...], (2, 0, 1, 3))
        ksend[...] = k_t
        vsend[...] = v_t
        krecv[pl.ds(my, 1)] = ksend[pl.ds(my * H_LOC, H_LOC)][None]
        vrecv[pl.ds(my, 1)] = vsend[pl.ds(my * H_LOC, H_LOC)][None]

        kv_rdmas = []
        for d in range(1, N_DEV):
            j = lax.rem(my + d, N_DEV)
            for send_buf, recv_buf, ssems, rsems in (
                (ksend, krecv, k_send_sems, k_recv_sems),
                (vsend, vrecv, v_send_sems, v_recv_sems),
            ):
                r = pltpu.make_async_remote_copy(
                    src_ref=send_buf.at[pl.ds(j * H_LOC, H_LOC)],
                    dst_ref=recv_buf.at[my],
                    send_sem=ssems.at[j],
                    recv_sem=rsems.at[my],
                    device_id=(j,),
                    device_id_type=pl.DeviceIdType.MESH,
                )
                r.start()
                kv_rdmas.append(r)

        wq = wq_ref[...]
        qs = [lax.dot(x_ref[b], wq, preferred_element_type=jnp.float32)
              for b in range(B)]

        qmod = lax.broadcasted_iota(jnp.int32, (SQ, SKV), 0) // 64 % 4
        kmod = lax.broadcasted_iota(jnp.int32, (SQ, SKV), 1) // 64 % 4
        mask = qmod == kmod

        for d in range(1, N_DEV):
            j = lax.rem(my + d, N_DEV)
            for send_buf, recv_buf, ssems, rsems in (
                (ksend, krecv, k_send_sems, k_recv_sems),
                (vsend, vrecv, v_send_sems, v_recv_sems),
            ):
                pltpu.make_async_remote_copy(
                    src_ref=send_buf.at[pl.ds(0, H_LOC)],
                    dst_ref=recv_buf.at[j],
                    send_sem=ssems.at[j],
                    recv_sem=rsems.at[j],
                    device_id=(j,),
                    device_id_type=pl.DeviceIdType.MESH,
                ).wait_recv()

        wo = wo_ref[...]
        partial_bs = []
        for b in range(B):
            head_ctx = []
            for h in range(H_LOC):
                q_bh = qs[b][:, h * DH:(h + 1) * DH]
                k_bh = jnp.concatenate(
                    [krecv[s, h, b] for s in range(N_DEV)], axis=0)
                v_bh = jnp.concatenate(
                    [vrecv[s, h, b] for s in range(N_DEV)], axis=0)
                s_ = lax.dot_general(
                    q_bh, k_bh, (((1,), (1,)), ((), ())),
                    preferred_element_type=jnp.float32) * 0.125
                s_ = jnp.where(mask, s_, -1e9)
                m_ = jnp.max(s_, axis=1, keepdims=True)
                e_ = jnp.exp(s_ - m_)
                p_ = e_ / jnp.sum(e_, axis=1, keepdims=True)
                head_ctx.append(
                    lax.dot(p_, v_bh, preferred_element_type=jnp.float32))
            ctx_b = jnp.concatenate(head_ctx, axis=1)
            partial_bs.append(
                lax.dot(ctx_b, wo, preferred_element_type=jnp.float32))
        partial = jnp.stack(partial_bs, axis=0)

        precv[pl.ds(my, 1)] = partial[None]
        p_rdmas = []
        for d in range(1, N_DEV):
            j = lax.rem(my + d, N_DEV)
            r = pltpu.make_async_remote_copy(
                src_ref=precv.at[my],
                dst_ref=precv.at[my],
                send_sem=p_send_sems.at[j],
                recv_sem=p_recv_sems.at[my],
                device_id=(j,),
                device_id_type=pl.DeviceIdType.MESH,
            )
            r.start()
            p_rdmas.append(r)
        for d in range(1, N_DEV):
            j = lax.rem(my + d, N_DEV)
            pltpu.make_async_remote_copy(
                src_ref=precv.at[my],
                dst_ref=precv.at[j],
                send_sem=p_send_sems.at[j],
                recv_sem=p_recv_sems.at[j],
                device_id=(j,),
                device_id_type=pl.DeviceIdType.MESH,
            ).wait_recv()

        out_ref[...] = jnp.sum(precv[...], axis=0)

        for r in kv_rdmas + p_rdmas:
            r.wait_send()

    return pl.pallas_call(
        body,
        out_shape=jax.ShapeDtypeStruct((B, SQ, D_MODEL), jnp.float32),
        in_specs=[pl.BlockSpec(memory_space=pltpu.VMEM)] * 5,
        out_specs=pl.BlockSpec(memory_space=pltpu.VMEM),
        scratch_shapes=[
            pltpu.VMEM((HQ, B, SKV_LOC, DH), jnp.float32),
            pltpu.VMEM((HQ, B, SKV_LOC, DH), jnp.float32),
            pltpu.VMEM((N_DEV, H_LOC, B, SKV_LOC, DH), jnp.float32),
            pltpu.VMEM((N_DEV, H_LOC, B, SKV_LOC, DH), jnp.float32),
            pltpu.VMEM((N_DEV, B, SQ, D_MODEL), jnp.float32),
            pltpu.SemaphoreType.DMA((N_DEV,)),
            pltpu.SemaphoreType.DMA((N_DEV,)),
            pltpu.SemaphoreType.DMA((N_DEV,)),
            pltpu.SemaphoreType.DMA((N_DEV,)),
            pltpu.SemaphoreType.DMA((N_DEV,)),
            pltpu.SemaphoreType.DMA((N_DEV,)),
        ],
        compiler_params=pltpu.CompilerParams(collective_id=0),
    )(x, Wq, K_ext, V_ext, Wo)


# device time: 53795 ns/iter; 1.6757x vs baseline; 1.6757x over previous
import jax
import jax.numpy as jnp
from jax import lax
from jax.experimental import pallas as pl
from jax.experimental.pallas import tpu as pltpu

N_DEV = 4
B, SQ, D_MODEL = 2, 256, 512
SKV = 1024
HQ, DH = 16, 64
H_LOC = HQ // N_DEV
SKV_LOC = SKV // N_DEV
SQ_C = SQ // N_DEV
BF16 = jnp.bfloat16


def kernel(x, Wq, K_ext, V_ext, Wo):
    def body(x_ref, wq_ref, k_ref, v_ref, wo_ref, out_ref,
             ksend, vsend, krecv, vrecv, pchunks, rsbuf, agbuf,
             k_send_sems, k_recv_sems, v_send_sems, v_recv_sems,
             rs_send_sems, rs_recv_sems, ag_send_sems, ag_recv_sems):
        my = lax.axis_index("i")

        barrier = pltpu.get_barrier_semaphore()
        for d in range(1, N_DEV):
            peer = lax.rem(my + d, N_DEV)
            pl.semaphore_signal(barrier, inc=1, device_id=(peer,),
                                device_id_type=pl.DeviceIdType.MESH)
        pl.semaphore_wait(barrier, N_DEV - 1)

        k_t = jnp.transpose(k_ref[...], (2, 0, 1, 3)).astype(BF16)
        v_t = jnp.transpose(v_ref[...], (2, 0, 1, 3)).astype(BF16)
        ksend[...] = k_t
        vsend[...] = v_t
        krecv[pl.ds(my, 1)] = ksend[pl.ds(my * H_LOC, H_LOC)][None]
        vrecv[pl.ds(my, 1)] = vsend[pl.ds(my * H_LOC, H_LOC)][None]

        kv_rdmas = []
        for d in range(1, N_DEV):
            j = lax.rem(my + d, N_DEV)
            for send_buf, recv_buf, ssems, rsems in (
                (ksend, krecv, k_send_sems, k_recv_sems),
                (vsend, vrecv, v_send_sems, v_recv_sems),
            ):
                r = pltpu.make_async_remote_copy(
                    src_ref=send_buf.at[pl.ds(j * H_LOC, H_LOC)],
                    dst_ref=recv_buf.at[my],
                    send_sem=ssems.at[j],
                    recv_sem=rsems.at[my],
                    device_id=(j,),
                    device_id_type=pl.DeviceIdType.MESH,
                )
                r.start()
                kv_rdmas.append(r)

        wq = wq_ref[...]
        qs = [lax.dot(x_ref[b], wq,
                      preferred_element_type=jnp.float32).astype(BF16)
              for b in range(B)]

        qmod = lax.broadcasted_iota(jnp.int32, (SQ, SKV), 0) // 64 % 4
        kmod = lax.broadcasted_iota(jnp.int32, (SQ, SKV), 1) // 64 % 4
        mask = qmod == kmod

        for d in range(1, N_DEV):
            j = lax.rem(my + d, N_DEV)
            for send_buf, recv_buf, ssems, rsems in (
                (ksend, krecv, k_send_sems, k_recv_sems),
                (vsend, vrecv, v_send_sems, v_recv_sems),
            ):
                pltpu.make_async_remote_copy(
                    src_ref=send_buf.at[pl.ds(0, H_LOC)],
                    dst_ref=recv_buf.at[j],
                    send_sem=ssems.at[j],
                    recv_sem=rsems.at[j],
                    device_id=(j,),
                    device_id_type=pl.DeviceIdType.MESH,
                ).wait_recv()

        wo = wo_ref[...]
        for b in range(B):
            head_ctx = []
            for h in range(H_LOC):
                q_bh = qs[b][:, h * DH:(h + 1) * DH]
                k_bh = jnp.concatenate(
                    [krecv[s, h, b] for s in range(N_DEV)], axis=0)
                v_bh = jnp.concatenate(
                    [vrecv[s, h, b] for s in range(N_DEV)], axis=0)
                s_ = lax.dot_general(
                    q_bh, k_bh, (((1,), (1,)), ((), ())),
                    preferred_element_type=jnp.float32) * 0.125
                s_ = jnp.where(mask, s_, -1e9)
                m_ = jnp.max(s_, axis=1, keepdims=True)
                e_ = jnp.exp(s_ - m_)
                p_ = (e_ / jnp.sum(e_, axis=1, keepdims=True)).astype(BF16)
                head_ctx.append(
                    lax.dot(p_, v_bh, preferred_element_type=jnp.float32))
            ctx_b = jnp.concatenate(head_ctx, axis=1)
            partial_b = lax.dot(ctx_b, wo,
                                preferred_element_type=jnp.float32)
            for c in range(N_DEV):
                pchunks[c, b] = partial_b[c * SQ_C:(c + 1) * SQ_C].astype(BF16)

        rsbuf[pl.ds(my, 1)] = pchunks[pl.ds(my, 1)]
        rs_rdmas = []
        for d in range(1, N_DEV):
            j = lax.rem(my + d, N_DEV)
            r = pltpu.make_async_remote_copy(
                src_ref=pchunks.at[j],
                dst_ref=rsbuf.at[my],
                send_sem=rs_send_sems.at[j],
                recv_sem=rs_recv_sems.at[my],
                device_id=(j,),
                device_id_type=pl.DeviceIdType.MESH,
            )
            r.start()
            rs_rdmas.append(r)
        for d in range(1, N_DEV):
            j = lax.rem(my + d, N_DEV)
            pltpu.make_async_remote_copy(
                src_ref=pchunks.at[j],
                dst_ref=rsbuf.at[j],
                send_sem=rs_send_sems.at[j],
                recv_sem=rs_recv_sems.at[j],
                device_id=(j,),
                device_id_type=pl.DeviceIdType.MESH,
            ).wait_recv()
        my_sum = jnp.sum(rsbuf[...].astype(jnp.float32), axis=0)
        agbuf[pl.ds(my, 1)] = my_sum.astype(BF16)[None]

        ag_rdmas = []
        for d in range(1, N_DEV):
            j = lax.rem(my + d, N_DEV)
            r = pltpu.make_async_remote_copy(
                src_ref=agbuf.at[my],
                dst_ref=agbuf.at[my],
                send_sem=ag_send_sems.at[j],
                recv_sem=ag_recv_sems.at[my],
                device_id=(j,),
                device_id_type=pl.DeviceIdType.MESH,
            )
            r.start()
            ag_rdmas.append(r)
        for d in range(1, N_DEV):
            j = lax.rem(my + d, N_DEV)
            pltpu.make_async_remote_copy(
                src_ref=agbuf.at[my],
                dst_ref=agbuf.at[j],
                send_sem=ag_send_sems.at[j],
                recv_sem=ag_recv_sems.at[j],
                device_id=(j,),
                device_id_type=pl.DeviceIdType.MESH,
            ).wait_recv()

        out_ref[...] = jnp.transpose(
            agbuf[...].astype(jnp.float32), (1, 0, 2, 3)
        ).reshape(B, SQ, D_MODEL)

        for r in kv_rdmas + rs_rdmas + ag_rdmas:
            r.wait_send()

    return pl.pallas_call(
        body,
        out_shape=jax.ShapeDtypeStruct((B, SQ, D_MODEL), jnp.float32),
        in_specs=[pl.BlockSpec(memory_space=pltpu.VMEM)] * 5,
        out_specs=pl.BlockSpec(memory_space=pltpu.VMEM),
        scratch_shapes=[
            pltpu.VMEM((HQ, B, SKV_LOC, DH), BF16),
            pltpu.VMEM((HQ, B, SKV_LOC, DH), BF16),
            pltpu.VMEM((N_DEV, H_LOC, B, SKV_LOC, DH), BF16),
            pltpu.VMEM((N_DEV, H_LOC, B, SKV_LOC, DH), BF16),
            pltpu.VMEM((N_DEV, B, SQ_C, D_MODEL), BF16),
            pltpu.VMEM((N_DEV, B, SQ_C, D_MODEL), BF16),
            pltpu.VMEM((N_DEV, B, SQ_C, D_MODEL), BF16),
            pltpu.SemaphoreType.DMA((N_DEV,)),
            pltpu.SemaphoreType.DMA((N_DEV,)),
            pltpu.SemaphoreType.DMA((N_DEV,)),
            pltpu.SemaphoreType.DMA((N_DEV,)),
            pltpu.SemaphoreType.DMA((N_DEV,)),
            pltpu.SemaphoreType.DMA((N_DEV,)),
            pltpu.SemaphoreType.DMA((N_DEV,)),
            pltpu.SemaphoreType.DMA((N_DEV,)),
        ],
        compiler_params=pltpu.CompilerParams(collective_id=0),
    )(x, Wq, K_ext, V_ext, Wo)
